# baseline (speedup 1.0000x reference)
import functools

import jax
import jax.numpy as jnp
from jax import lax
from jax.experimental import pallas as pl
from jax.experimental.pallas import tpu as pltpu
from jax.experimental.pallas import tpu_sc as plsc

_B, _S, _D = 4, 8192, 768
_R_SC = 2048
_NC, _NS = 2, 16
_NW = _NC * _NS
_ROWS_W = _R_SC // _NW
_CS = 16
_NCHUNK = _ROWS_W // _CS
_L = 16
_NJ = _D // _L

_BLOCK_S = 1024


def _sc_body(in_hbm, tab_hbm, out_hbm, x_v, t_v, si0, si1, so0, so1):
    wid = lax.axis_index("s") * _NC + lax.axis_index("c")
    base = wid * _ROWS_W
    sin = (si0, si1)
    sout = (so0, so1)

    def issue_in(s0, buf):
        cps = [pltpu.async_copy(tab_hbm.at[pl.ds(s0, _CS)], t_v.at[buf], sin[buf])]
        for b in range(_B):
            cps.append(
                pltpu.async_copy(in_hbm.at[b, pl.ds(s0, _CS)], x_v.at[buf, b], sin[buf])
            )
        return cps

    def issue_out(s0, buf):
        return [
            pltpu.async_copy(x_v.at[buf, b], out_hbm.at[b, pl.ds(s0, _CS)], sout[buf])
            for b in range(_B)
        ]

    def compute(buf):
        def row(r, c2):
            for j in range(_NJ):
                sl = pl.ds(j * _L, _L)
                tv = t_v[buf, r, sl]
                for b in range(_B):
                    x_v[buf, b, r, sl] = x_v[buf, b, r, sl] + tv
            return c2

        lax.fori_loop(0, _CS, row, 0)

    def pair(g, carry):
        s0 = base + g * 2 * _CS
        s1 = s0 + _CS
        i0 = issue_in(s0, 0)
        i1 = issue_in(s1, 1)
        for cp in i0:
            cp.wait()
        compute(0)
        o0 = issue_out(s0, 0)
        for cp in i1:
            cp.wait()
        compute(1)
        o1 = issue_out(s1, 1)
        for cp in o0:
            cp.wait()
        for cp in o1:
            cp.wait()
        return carry

    lax.fori_loop(0, _NCHUNK // 2, pair, 0)


_sc_kernel = functools.partial(
    pl.kernel,
    mesh=plsc.VectorSubcoreMesh(core_axis_name="c", subcore_axis_name="s"),
    out_type=jax.ShapeDtypeStruct((_B, _R_SC, _D), jnp.float32),
    scratch_types=[
        pltpu.VMEM((2, _B, _CS, _D), jnp.float32),
        pltpu.VMEM((2, _CS, _D), jnp.float32),
        pltpu.SemaphoreType.DMA,
        pltpu.SemaphoreType.DMA,
        pltpu.SemaphoreType.DMA,
        pltpu.SemaphoreType.DMA,
    ],
)(_sc_body)


def _tc_body(x_ref, t_ref, o_ref):
    o_ref[...] = x_ref[...] + t_ref[...][None, :, :]


def _tc_tail(inputs, pos_emb_table):
    off = _R_SC // _BLOCK_S
    return pl.pallas_call(
        _tc_body,
        grid=((_S - _R_SC) // _BLOCK_S,),
        in_specs=[
            pl.BlockSpec((_B, _BLOCK_S, _D), lambda i: (0, i + off, 0)),
            pl.BlockSpec((_BLOCK_S, _D), lambda i: (i + off, 0)),
        ],
        out_specs=pl.BlockSpec((_B, _BLOCK_S, _D), lambda i: (0, i + off, 0)),
        out_shape=jax.ShapeDtypeStruct((_B, _S, _D), inputs.dtype),
    )(inputs, pos_emb_table)


def kernel(inputs, pos_emb_table):
    head = _sc_kernel(inputs, pos_emb_table)
    tail = _tc_tail(inputs, pos_emb_table)
    return lax.dynamic_update_slice(tail, head, (0, 0, 0))


# speedup vs baseline: 1.5562x; 1.5562x over previous
import jax
import jax.numpy as jnp
from jax.experimental import pallas as pl

_BLOCK_S = 256


def _add_body(x_ref, t_ref, o_ref):
    o_ref[...] = x_ref[...] + t_ref[...][None, :, :]


def kernel(inputs, pos_emb_table):
    B, S, D = inputs.shape
    return pl.pallas_call(
        _add_body,
        grid=(S // _BLOCK_S,),
        in_specs=[
            pl.BlockSpec((B, _BLOCK_S, D), lambda i: (0, i, 0)),
            pl.BlockSpec((_BLOCK_S, D), lambda i: (i, 0)),
        ],
        out_specs=pl.BlockSpec((B, _BLOCK_S, D), lambda i: (0, i, 0)),
        out_shape=jax.ShapeDtypeStruct((B, S, D), inputs.dtype),
    )(inputs, pos_emb_table)


# speedup vs baseline: 1.5925x; 1.0233x over previous
import jax
import jax.numpy as jnp
from jax.experimental import pallas as pl

_BLOCK_S = 1024


def _add_body(x_ref, t_ref, o_ref):
    o_ref[...] = x_ref[...] + t_ref[...][None, :, :]


def kernel(inputs, pos_emb_table):
    B, S, D = inputs.shape
    return pl.pallas_call(
        _add_body,
        grid=(S // _BLOCK_S,),
        in_specs=[
            pl.BlockSpec((B, _BLOCK_S, D), lambda i: (0, i, 0)),
            pl.BlockSpec((_BLOCK_S, D), lambda i: (i, 0)),
        ],
        out_specs=pl.BlockSpec((B, _BLOCK_S, D), lambda i: (0, i, 0)),
        out_shape=jax.ShapeDtypeStruct((B, S, D), inputs.dtype),
    )(inputs, pos_emb_table)


# speedup vs baseline: 1.6005x; 1.0050x over previous
import jax
import jax.numpy as jnp
from jax.experimental import pallas as pl
from jax.experimental.pallas import tpu as pltpu

_BLOCK_S = 512


def _add_body(x_ref, t_ref, o_ref):
    o_ref[...] = x_ref[...] + t_ref[...][None, :, :]


def kernel(inputs, pos_emb_table):
    B, S, D = inputs.shape
    return pl.pallas_call(
        _add_body,
        grid=(S // _BLOCK_S,),
        in_specs=[
            pl.BlockSpec((B, _BLOCK_S, D), lambda i: (0, i, 0)),
            pl.BlockSpec((_BLOCK_S, D), lambda i: (i, 0)),
        ],
        out_specs=pl.BlockSpec((B, _BLOCK_S, D), lambda i: (0, i, 0)),
        out_shape=jax.ShapeDtypeStruct((B, S, D), inputs.dtype),
        compiler_params=pltpu.CompilerParams(
            dimension_semantics=("parallel",)
        ),
    )(inputs, pos_emb_table)
